# baseline (speedup 1.0000x reference)
import functools

import jax
import jax.numpy as jnp
from jax import lax
from jax.experimental import pallas as pl
from jax.experimental.pallas import tpu as pltpu
from jax.experimental.pallas import tpu_sc as plsc

H = 12
LS = 16
BQ = 256
S = 2048
DH = 64
NW = 32
HG = 6
UNITS_PER_W = (HG * S // 128) // NW
TOKS_PER_W = UNITS_PER_W * 128


def _qkv_proj_kernel(x_ref, w_ref, b_ref, o_ref):
    o_ref[...] = (
        jnp.dot(x_ref[...], w_ref[...], preferred_element_type=jnp.float32)
        + b_ref[...]
    )


def _select_kernel(q_ref, o_ref):
    i = pl.program_id(1)
    qf = q_ref[0]
    qb = q_ref[0, pl.ds(i * BQ, BQ), :]
    sq_all = jnp.sum(qf * qf, axis=1)
    sq_blk = jnp.sum(qb * qb, axis=1)
    cross = jnp.dot(qb, qf.T, preferred_element_type=jnp.float32)
    d = sq_blk[:, None] + sq_all[None, :] - 2.0 * cross
    BIG = jnp.float32(3.0e38)
    iota = lax.broadcasted_iota(jnp.int32, (S, 1), 0)
    lohi = jnp.concatenate(
        [(iota % 64).astype(jnp.float32), (iota // 64).astype(jnp.float32)],
        axis=1)
    cols = []
    for _ in range(LS):
        mn = jnp.min(d, axis=1, keepdims=True)
        eq = d == mn
        eqb = jnp.where(eq, 1.0, 0.0)
        lh = jnp.dot(eqb, lohi, preferred_element_type=jnp.float32)
        cols.append(lh[:, 0:1] + 64.0 * lh[:, 1:2])
        d = jnp.where(eq, BIG, d)
    idx_f = jnp.concatenate(cols, axis=1)
    idx_f = jnp.clip(idx_f, 0.0, float(S - 1))
    o_ref[0] = (idx_f + 0.5).astype(jnp.int32)


def _out_proj_kernel(y_ref, w_ref, b_ref, x_ref, o_ref):
    o_ref[...] = (
        jnp.dot(y_ref[...], w_ref[...], preferred_element_type=jnp.float32)
        + b_ref[...]
        + x_ref[...]
    )


def _sc_attn_body(kh_ref, vh_ref, q_ref, idx_ref, out_ref,
                  table, qbuf, idxbuf, wscr, obuf):
    wid = lax.axis_index("s") * 2 + lax.axis_index("c")
    u0 = wid * UNITS_PER_W
    hA = u0 // 16
    hB = (u0 + UNITS_PER_W - 1) // 16
    split = jnp.minimum(UNITS_PER_W, (hA + 1) * 16 - u0)
    iota16 = lax.iota(jnp.int32, 16)
    scale = jnp.float32(DH ** (-0.5))

    def run_group(h, ulo, uhi):
        for half in (0, 1):
            pltpu.sync_copy(kh_ref.at[h, half], table)

            def unit_body(u, _):
                blk = (u0 + u) % 16
                pltpu.sync_copy(q_ref.at[h, pl.ds(blk * 8320, 8320)], qbuf)
                pltpu.sync_copy(idx_ref.at[h, pl.ds(blk * 2176, 2176)], idxbuf)

                def g_body(g, _):
                    rows = g * 16 + iota16
                    qrow0 = rows * 65 + (half * 32)
                    loff = u * 128 + g * 16
                    base = [plsc.load_gather(idxbuf, [rows * 17 + j]) * 33
                            for j in range(LS)]
                    if half == 0:
                        acc0 = tuple(jnp.zeros((16,), jnp.float32)
                                     for _ in range(LS))
                    else:
                        acc0 = tuple(wscr[pl.ds(j * TOKS_PER_W + loff, 16)]
                                     for j in range(LS))

                    def c_body(c, accs):
                        qc = plsc.load_gather(qbuf, [qrow0 + c])
                        return tuple(
                            accs[j] + plsc.load_gather(
                                table, [base[j] + c]) * qc
                            for j in range(LS))

                    accs = lax.fori_loop(0, 32, c_body, acc0)
                    for j in range(LS):
                        wscr[pl.ds(j * TOKS_PER_W + loff, 16)] = accs[j]
                    return 0

                lax.fori_loop(0, 8, g_body, 0)
                return 0

            lax.fori_loop(ulo, uhi, unit_body, 0)

        def sm_body(u, _):
            def sm_g(g, _):
                loff = u * 128 + g * 16
                lg = [wscr[pl.ds(j * TOKS_PER_W + loff, 16)] for j in range(LS)]
                m = lg[0]
                for j in range(1, LS):
                    m = jnp.maximum(m, lg[j])
                e = [jnp.exp((lg[j] - m) * scale) for j in range(LS)]
                ssum = e[0]
                for j in range(1, LS):
                    ssum = ssum + e[j]
                inv = jnp.float32(1.0) / ssum
                for j in range(LS):
                    wscr[pl.ds(j * TOKS_PER_W + loff, 16)] = e[j] * inv
                return 0
            lax.fori_loop(0, 8, sm_g, 0)
            return 0

        lax.fori_loop(ulo, uhi, sm_body, 0)

        for half in (0, 1):
            pltpu.sync_copy(vh_ref.at[h, half], table)

            def vunit_body(u, _):
                blk = (u0 + u) % 16
                pltpu.sync_copy(idx_ref.at[h, pl.ds(blk * 2176, 2176)], idxbuf)

                def vg_body(g, _):
                    rows = g * 16 + iota16
                    base = [plsc.load_gather(idxbuf, [rows * 17 + j]) * 33
                            for j in range(LS)]
                    loff = u * 128 + g * 16
                    w = [wscr[pl.ds(j * TOKS_PER_W + loff, 16)] for j in range(LS)]
                    orow = rows * 33

                    def vc_body(c, _):
                        acc = w[0] * plsc.load_gather(table, [base[0] + c])
                        for j in range(1, LS):
                            acc = acc + w[j] * plsc.load_gather(
                                table, [base[j] + c])
                        plsc.store_scatter(obuf, [orow + c], acc)
                        return 0

                    lax.fori_loop(0, 32, vc_body, 0)
                    return 0

                lax.fori_loop(0, 8, vg_body, 0)
                pltpu.sync_copy(obuf,
                                out_ref.at[h, half, pl.ds(blk * 4224, 4224)])
                return 0

            lax.fori_loop(ulo, uhi, vunit_body, 0)

    run_group(hA, 0, split)

    @pl.when(split < UNITS_PER_W)
    def _():
        run_group(hB, split, UNITS_PER_W)


def _sc_attention(khalves, vhalves, qflat, idxflat):
    mesh = plsc.VectorSubcoreMesh(core_axis_name="c", subcore_axis_name="s")
    return pl.kernel(
        _sc_attn_body,
        mesh=mesh,
        compiler_params=pltpu.CompilerParams(needs_layout_passes=False),
        out_type=jax.ShapeDtypeStruct((HG, 2, S * 33), jnp.float32),
        scratch_types=[
            pltpu.VMEM((S * 33,), jnp.float32),
            pltpu.VMEM((128 * 65,), jnp.float32),
            pltpu.VMEM((128 * 17,), jnp.int32),
            pltpu.VMEM((TOKS_PER_W * 16,), jnp.float32),
            pltpu.VMEM((128 * 33,), jnp.float32),
        ],
    )(khalves, vhalves, qflat, idxflat)


def kernel(x, W_qkv, b_qkv, W_proj, b_proj):
    B, _, D = x.shape
    x2 = x.reshape(S, D)

    qkv = pl.pallas_call(
        _qkv_proj_kernel,
        grid=(S // BQ,),
        in_specs=[
            pl.BlockSpec((BQ, D), lambda i: (i, 0)),
            pl.BlockSpec((D, 3 * D), lambda i: (0, 0)),
            pl.BlockSpec((3 * D,), lambda i: (0,)),
        ],
        out_specs=pl.BlockSpec((BQ, 3 * D), lambda i: (i, 0)),
        out_shape=jax.ShapeDtypeStruct((S, 3 * D), jnp.float32),
    )(x2, W_qkv, b_qkv)

    qkv = qkv.reshape(S, 3, H, DH).transpose(1, 2, 0, 3)
    q, k, v = qkv[0], qkv[1], qkv[2]

    pad1 = ((0, 0), (0, 0), (0, 0), (0, 1))
    khalves = jnp.pad(k.reshape(H, S, 2, 32).transpose(0, 2, 1, 3), pad1
                      ).reshape(H, 2, S * 33)
    vhalves = jnp.pad(v.reshape(H, S, 2, 32).transpose(0, 2, 1, 3), pad1
                      ).reshape(H, 2, S * 33)
    qflat = jnp.pad(q, ((0, 0), (0, 0), (0, 1))).reshape(H, S * 65)

    def select_group(qg):
        return pl.pallas_call(
            _select_kernel,
            grid=(HG, S // BQ),
            in_specs=[pl.BlockSpec((1, S, DH), lambda h, i: (h, 0, 0))],
            out_specs=pl.BlockSpec((1, BQ, LS), lambda h, i: (h, i, 0)),
            out_shape=jax.ShapeDtypeStruct((HG, S, LS), jnp.int32),
        )(qg)

    outs = []
    for g0 in (0, HG):
        idx_g = select_group(q[g0:g0 + HG])
        idxflat_g = jnp.pad(idx_g, ((0, 0), (0, 0), (0, 1))).reshape(HG, S * 17)
        outs.append(_sc_attention(khalves[g0:g0 + HG], vhalves[g0:g0 + HG],
                                  qflat[g0:g0 + HG], idxflat_g))
    out_sc = jnp.concatenate(outs, axis=0)

    y = (out_sc.reshape(H, 2, S, 33)[..., :32]
         .transpose(2, 0, 1, 3).reshape(S, D))

    res = pl.pallas_call(
        _out_proj_kernel,
        grid=(S // BQ,),
        in_specs=[
            pl.BlockSpec((BQ, D), lambda i: (i, 0)),
            pl.BlockSpec((D, D), lambda i: (0, 0)),
            pl.BlockSpec((D,), lambda i: (0,)),
            pl.BlockSpec((BQ, D), lambda i: (i, 0)),
        ],
        out_specs=pl.BlockSpec((BQ, D), lambda i: (i, 0)),
        out_shape=jax.ShapeDtypeStruct((S, D), jnp.float32),
    )(y, W_proj, b_proj, x2)

    return res.reshape(B, S, D)


# speedup vs baseline: 1.0220x; 1.0220x over previous
import functools

import jax
import jax.numpy as jnp
from jax import lax
from jax.experimental import pallas as pl
from jax.experimental.pallas import tpu as pltpu
from jax.experimental.pallas import tpu_sc as plsc

H = 12
LS = 16
BQ = 256
S = 2048
DH = 64
NW = 32
HG = 12
UNITS_PER_W = (HG * S // 128) // NW
TOKS_PER_W = UNITS_PER_W * 128


def _qkv_proj_kernel(x_ref, w_ref, b_ref, o_ref):
    o_ref[...] = (
        jnp.dot(x_ref[...], w_ref[...], preferred_element_type=jnp.float32)
        + b_ref[...]
    )


def _select_kernel(q_ref, o_ref):
    i = pl.program_id(1)
    qf = q_ref[0]
    qb = q_ref[0, pl.ds(i * BQ, BQ), :]
    sq_all = jnp.sum(qf * qf, axis=1)
    sq_blk = jnp.sum(qb * qb, axis=1)
    cross = jnp.dot(qb, qf.T, preferred_element_type=jnp.float32)
    d = sq_blk[:, None] + sq_all[None, :] - 2.0 * cross
    BIG = jnp.float32(3.0e38)
    iota = lax.broadcasted_iota(jnp.int32, (S, 1), 0)
    lohi = jnp.concatenate(
        [(iota % 64).astype(jnp.float32), (iota // 64).astype(jnp.float32)],
        axis=1)
    cols = []
    for _ in range(LS):
        mn = jnp.min(d, axis=1, keepdims=True)
        eq = d == mn
        eqb = jnp.where(eq, 1.0, 0.0)
        lh = jnp.dot(eqb, lohi, preferred_element_type=jnp.float32)
        cols.append(lh[:, 0:1] + 64.0 * lh[:, 1:2])
        d = jnp.where(eq, BIG, d)
    idx_f = jnp.concatenate(cols, axis=1)
    idx_f = jnp.clip(idx_f, 0.0, float(S - 1))
    o_ref[0] = (idx_f + 0.5).astype(jnp.int32)


def _out_proj_kernel(y_ref, w_ref, b_ref, x_ref, o_ref):
    o_ref[...] = (
        jnp.dot(y_ref[...], w_ref[...], preferred_element_type=jnp.float32)
        + b_ref[...]
        + x_ref[...]
    )


def _sc_attn_body(kh_ref, vh_ref, q_ref, idx_ref, out_ref,
                  table, qbuf, idxbuf, wscr, obuf):
    wid = lax.axis_index("s") * 2 + lax.axis_index("c")
    u0 = wid * UNITS_PER_W
    hA = u0 // 16
    hB = (u0 + UNITS_PER_W - 1) // 16
    split = jnp.minimum(UNITS_PER_W, (hA + 1) * 16 - u0)
    iota16 = lax.iota(jnp.int32, 16)
    scale = jnp.float32(DH ** (-0.5))

    def run_group(h, ulo, uhi):
        for half in (0, 1):
            pltpu.sync_copy(kh_ref.at[h, half], table)

            def unit_body(u, _):
                blk = (u0 + u) % 16
                pltpu.sync_copy(q_ref.at[h, pl.ds(blk * 8320, 8320)], qbuf)
                pltpu.sync_copy(idx_ref.at[h, pl.ds(blk * 2176, 2176)], idxbuf)

                def g_body(g, _):
                    rows = g * 16 + iota16
                    qrow0 = rows * 65 + (half * 32)
                    loff = u * 128 + g * 16
                    base = [plsc.load_gather(idxbuf, [rows * 17 + j]) * 33
                            for j in range(LS)]
                    if half == 0:
                        acc0 = tuple(jnp.zeros((16,), jnp.float32)
                                     for _ in range(LS))
                    else:
                        acc0 = tuple(wscr[pl.ds(j * TOKS_PER_W + loff, 16)]
                                     for j in range(LS))

                    def c_body(c, accs):
                        qc = plsc.load_gather(qbuf, [qrow0 + c])
                        return tuple(
                            accs[j] + plsc.load_gather(
                                table, [base[j] + c]) * qc
                            for j in range(LS))

                    accs = lax.fori_loop(0, 32, c_body, acc0)
                    for j in range(LS):
                        wscr[pl.ds(j * TOKS_PER_W + loff, 16)] = accs[j]
                    return 0

                lax.fori_loop(0, 8, g_body, 0)
                return 0

            lax.fori_loop(ulo, uhi, unit_body, 0)

        def sm_body(u, _):
            def sm_g(g, _):
                loff = u * 128 + g * 16
                lg = [wscr[pl.ds(j * TOKS_PER_W + loff, 16)] for j in range(LS)]
                m = lg[0]
                for j in range(1, LS):
                    m = jnp.maximum(m, lg[j])
                e = [jnp.exp((lg[j] - m) * scale) for j in range(LS)]
                ssum = e[0]
                for j in range(1, LS):
                    ssum = ssum + e[j]
                inv = jnp.float32(1.0) / ssum
                for j in range(LS):
                    wscr[pl.ds(j * TOKS_PER_W + loff, 16)] = e[j] * inv
                return 0
            lax.fori_loop(0, 8, sm_g, 0)
            return 0

        lax.fori_loop(ulo, uhi, sm_body, 0)

        for half in (0, 1):
            pltpu.sync_copy(vh_ref.at[h, half], table)

            def vunit_body(u, _):
                blk = (u0 + u) % 16
                pltpu.sync_copy(idx_ref.at[h, pl.ds(blk * 2176, 2176)], idxbuf)

                def vg_body(g, _):
                    rows = g * 16 + iota16
                    base = [plsc.load_gather(idxbuf, [rows * 17 + j]) * 33
                            for j in range(LS)]
                    loff = u * 128 + g * 16
                    w = [wscr[pl.ds(j * TOKS_PER_W + loff, 16)] for j in range(LS)]
                    orow = rows * 33

                    def vc_body(c, _):
                        acc = w[0] * plsc.load_gather(table, [base[0] + c])
                        for j in range(1, LS):
                            acc = acc + w[j] * plsc.load_gather(
                                table, [base[j] + c])
                        plsc.store_scatter(obuf, [orow + c], acc)
                        return 0

                    lax.fori_loop(0, 32, vc_body, 0)
                    return 0

                lax.fori_loop(0, 8, vg_body, 0)
                pltpu.sync_copy(obuf,
                                out_ref.at[h, half, pl.ds(blk * 4224, 4224)])
                return 0

            lax.fori_loop(ulo, uhi, vunit_body, 0)

    run_group(hA, 0, split)

    @pl.when(split < UNITS_PER_W)
    def _():
        run_group(hB, split, UNITS_PER_W)


def _sc_attention(khalves, vhalves, qflat, idxflat):
    mesh = plsc.VectorSubcoreMesh(core_axis_name="c", subcore_axis_name="s")
    return pl.kernel(
        _sc_attn_body,
        mesh=mesh,
        compiler_params=pltpu.CompilerParams(needs_layout_passes=False),
        out_type=jax.ShapeDtypeStruct((HG, 2, S * 33), jnp.float32),
        scratch_types=[
            pltpu.VMEM((S * 33,), jnp.float32),
            pltpu.VMEM((128 * 65,), jnp.float32),
            pltpu.VMEM((128 * 17,), jnp.int32),
            pltpu.VMEM((TOKS_PER_W * 16,), jnp.float32),
            pltpu.VMEM((128 * 33,), jnp.float32),
        ],
    )(khalves, vhalves, qflat, idxflat)


def kernel(x, W_qkv, b_qkv, W_proj, b_proj):
    B, _, D = x.shape
    x2 = x.reshape(S, D)

    qkv = pl.pallas_call(
        _qkv_proj_kernel,
        grid=(S // BQ,),
        in_specs=[
            pl.BlockSpec((BQ, D), lambda i: (i, 0)),
            pl.BlockSpec((D, 3 * D), lambda i: (0, 0)),
            pl.BlockSpec((3 * D,), lambda i: (0,)),
        ],
        out_specs=pl.BlockSpec((BQ, 3 * D), lambda i: (i, 0)),
        out_shape=jax.ShapeDtypeStruct((S, 3 * D), jnp.float32),
    )(x2, W_qkv, b_qkv)

    qkv = qkv.reshape(S, 3, H, DH).transpose(1, 2, 0, 3)
    q, k, v = qkv[0], qkv[1], qkv[2]

    pad1 = ((0, 0), (0, 0), (0, 0), (0, 1))
    khalves = jnp.pad(k.reshape(H, S, 2, 32).transpose(0, 2, 1, 3), pad1
                      ).reshape(H, 2, S * 33)
    vhalves = jnp.pad(v.reshape(H, S, 2, 32).transpose(0, 2, 1, 3), pad1
                      ).reshape(H, 2, S * 33)
    qflat = jnp.pad(q, ((0, 0), (0, 0), (0, 1))).reshape(H, S * 65)

    def select_group(qg):
        return pl.pallas_call(
            _select_kernel,
            grid=(HG, S // BQ),
            in_specs=[pl.BlockSpec((1, S, DH), lambda h, i: (h, 0, 0))],
            out_specs=pl.BlockSpec((1, BQ, LS), lambda h, i: (h, i, 0)),
            out_shape=jax.ShapeDtypeStruct((HG, S, LS), jnp.int32),
        )(qg)

    outs = []
    for g0 in range(0, H, HG):
        idx_g = select_group(q[g0:g0 + HG])
        idxflat_g = jnp.pad(idx_g, ((0, 0), (0, 0), (0, 1))).reshape(HG, S * 17)
        outs.append(_sc_attention(khalves[g0:g0 + HG], vhalves[g0:g0 + HG],
                                  qflat[g0:g0 + HG], idxflat_g))
    out_sc = jnp.concatenate(outs, axis=0)

    y = (out_sc.reshape(H, 2, S, 33)[..., :32]
         .transpose(2, 0, 1, 3).reshape(S, D))

    res = pl.pallas_call(
        _out_proj_kernel,
        grid=(S // BQ,),
        in_specs=[
            pl.BlockSpec((BQ, D), lambda i: (i, 0)),
            pl.BlockSpec((D, D), lambda i: (0, 0)),
            pl.BlockSpec((D,), lambda i: (0,)),
            pl.BlockSpec((BQ, D), lambda i: (i, 0)),
        ],
        out_specs=pl.BlockSpec((BQ, D), lambda i: (i, 0)),
        out_shape=jax.ShapeDtypeStruct((S, D), jnp.float32),
    )(y, W_proj, b_proj, x2)

    return res.reshape(B, S, D)


# speedup vs baseline: 1.0224x; 1.0003x over previous
import jax
import jax.numpy as jnp
from jax import lax
from jax.experimental import pallas as pl
from jax.experimental.pallas import tpu as pltpu
from jax.experimental.pallas import tpu_sc as plsc

H = 12
LS = 16
BQ = 256
S = 2048
DH = 64
NW = 32
HG = 12
UNITS_PER_W = (HG * S // 128) // NW
TOKS_PER_W = UNITS_PER_W * 128


def _qkv_proj_kernel(x_ref, w_ref, b_ref, o_ref):
    o_ref[...] = (
        jnp.dot(x_ref[...], w_ref[...], preferred_element_type=jnp.float32)
        + b_ref[...]
    )


def _select_kernel(q_ref, o_ref):
    i = pl.program_id(1)
    qf = q_ref[0]
    qb = q_ref[0, pl.ds(i * BQ, BQ), :]
    sq_all = jnp.sum(qf * qf, axis=1)
    sq_blk = jnp.sum(qb * qb, axis=1)
    cross = jnp.dot(qb, qf.T, preferred_element_type=jnp.float32)
    d = sq_blk[:, None] + sq_all[None, :] - 2.0 * cross
    BIG = jnp.float32(3.0e38)
    iota = lax.broadcasted_iota(jnp.int32, (S, 1), 0)
    lohi = jnp.concatenate(
        [(iota % 64).astype(jnp.float32), (iota // 64).astype(jnp.float32)],
        axis=1)
    cols = []
    for _ in range(LS):
        mn = jnp.min(d, axis=1, keepdims=True)
        eq = d == mn
        eqb = jnp.where(eq, 1.0, 0.0)
        lh = jnp.dot(eqb, lohi, preferred_element_type=jnp.float32)
        cols.append(lh[:, 0:1] + 64.0 * lh[:, 1:2])
        d = jnp.where(eq, BIG, d)
    idx_f = jnp.concatenate(cols, axis=1)
    idx_f = jnp.clip(idx_f, 0.0, float(S - 1))
    o_ref[0] = (idx_f + 0.5).astype(jnp.int32)


def _out_proj_kernel(y_ref, w_ref, b_ref, x_ref, o_ref):
    o_ref[...] = (
        jnp.dot(y_ref[...], w_ref[...], preferred_element_type=jnp.float32)
        + b_ref[...]
        + x_ref[...]
    )


def _sc_attn_body(kh_ref, vh_ref, q_ref, idx_ref, out_ref,
                  table, qbuf, idxbuf, wscr, obuf):
    wid = lax.axis_index("s") * 2 + lax.axis_index("c")
    u0 = wid * UNITS_PER_W
    hA = u0 // 16
    hB = (u0 + UNITS_PER_W - 1) // 16
    split = jnp.minimum(UNITS_PER_W, (hA + 1) * 16 - u0)
    iota16 = lax.iota(jnp.int32, 16)
    scale = jnp.float32(DH ** (-0.5))

    def run_group(h, ulo, uhi):
        for half in (0, 1):
            pltpu.sync_copy(kh_ref.at[h, half], table)

            def unit_body(u, _):
                blk = (u0 + u) % 16
                pltpu.sync_copy(q_ref.at[h, pl.ds(blk * 8320, 8320)], qbuf)
                pltpu.sync_copy(idx_ref.at[h, pl.ds(blk * 2176, 2176)], idxbuf)

                def g_body(g, _):
                    rows = g * 16 + iota16
                    qrow0 = rows * 65 + (half * 32)
                    loff = u * 128 + g * 16
                    base = [plsc.load_gather(idxbuf, [rows * 17 + j]) * 33
                            for j in range(LS)]
                    if half == 0:
                        acc0 = tuple(jnp.zeros((16,), jnp.float32)
                                     for _ in range(LS))
                    else:
                        acc0 = tuple(wscr[pl.ds(j * TOKS_PER_W + loff, 16)]
                                     for j in range(LS))

                    def c_body(c, accs):
                        qc = plsc.load_gather(qbuf, [qrow0 + c])
                        return tuple(
                            accs[j] + plsc.load_gather(
                                table, [base[j] + c]) * qc
                            for j in range(LS))

                    accs = lax.fori_loop(0, 32, c_body, acc0)
                    for j in range(LS):
                        wscr[pl.ds(j * TOKS_PER_W + loff, 16)] = accs[j]
                    return 0

                lax.fori_loop(0, 8, g_body, 0)
                return 0

            lax.fori_loop(ulo, uhi, unit_body, 0)

        def sm_body(u, _):
            def sm_g(g, _):
                loff = u * 128 + g * 16
                lg = [wscr[pl.ds(j * TOKS_PER_W + loff, 16)] for j in range(LS)]
                m = lg[0]
                for j in range(1, LS):
                    m = jnp.maximum(m, lg[j])
                e = [jnp.exp((lg[j] - m) * scale) for j in range(LS)]
                ssum = e[0]
                for j in range(1, LS):
                    ssum = ssum + e[j]
                inv = jnp.float32(1.0) / ssum
                for j in range(LS):
                    wscr[pl.ds(j * TOKS_PER_W + loff, 16)] = e[j] * inv
                return 0
            lax.fori_loop(0, 8, sm_g, 0)
            return 0

        lax.fori_loop(ulo, uhi, sm_body, 0)

        for half in (0, 1):
            pltpu.sync_copy(vh_ref.at[h, half], table)

            def vunit_body(u, _):
                blk = (u0 + u) % 16
                pltpu.sync_copy(idx_ref.at[h, pl.ds(blk * 2176, 2176)], idxbuf)

                def vg_body(g, _):
                    rows = g * 16 + iota16
                    base = [plsc.load_gather(idxbuf, [rows * 17 + j]) * 33
                            for j in range(LS)]
                    loff = u * 128 + g * 16
                    w = [wscr[pl.ds(j * TOKS_PER_W + loff, 16)] for j in range(LS)]
                    orow = rows * 33

                    def vc_body(c, _):
                        acc = w[0] * plsc.load_gather(table, [base[0] + c])
                        for j in range(1, LS):
                            acc = acc + w[j] * plsc.load_gather(
                                table, [base[j] + c])
                        plsc.store_scatter(obuf, [orow + c], acc)
                        return 0

                    lax.fori_loop(0, 32, vc_body, 0)
                    return 0

                lax.fori_loop(0, 8, vg_body, 0)
                pltpu.sync_copy(obuf,
                                out_ref.at[h, half, pl.ds(blk * 4224, 4224)])
                return 0

            lax.fori_loop(ulo, uhi, vunit_body, 0)

    run_group(hA, 0, split)

    @pl.when(split < UNITS_PER_W)
    def _():
        run_group(hB, split, UNITS_PER_W)


def _sc_attention(khalves, vhalves, qflat, idxflat):
    mesh = plsc.VectorSubcoreMesh(core_axis_name="c", subcore_axis_name="s")
    return pl.kernel(
        _sc_attn_body,
        mesh=mesh,
        compiler_params=pltpu.CompilerParams(needs_layout_passes=False),
        out_type=jax.ShapeDtypeStruct((HG, 2, S * 33), jnp.float32),
        scratch_types=[
            pltpu.VMEM((S * 33,), jnp.float32),
            pltpu.VMEM((128 * 65,), jnp.float32),
            pltpu.VMEM((128 * 17,), jnp.int32),
            pltpu.VMEM((TOKS_PER_W * 16,), jnp.float32),
            pltpu.VMEM((128 * 33,), jnp.float32),
        ],
    )(khalves, vhalves, qflat, idxflat)


def kernel(x, W_qkv, b_qkv, W_proj, b_proj):
    B, _, D = x.shape
    x2 = x.reshape(S, D)

    qkv = pl.pallas_call(
        _qkv_proj_kernel,
        grid=(S // BQ,),
        in_specs=[
            pl.BlockSpec((BQ, D), lambda i: (i, 0)),
            pl.BlockSpec((D, 3 * D), lambda i: (0, 0)),
            pl.BlockSpec((3 * D,), lambda i: (0,)),
        ],
        out_specs=pl.BlockSpec((BQ, 3 * D), lambda i: (i, 0)),
        out_shape=jax.ShapeDtypeStruct((S, 3 * D), jnp.float32),
    )(x2, W_qkv, b_qkv)

    qkv = qkv.reshape(S, 3, H, DH).transpose(1, 2, 0, 3)
    q, k, v = qkv[0], qkv[1], qkv[2]

    pad1 = ((0, 0), (0, 0), (0, 0), (0, 1))
    khalves = jnp.pad(k.reshape(H, S, 2, 32).transpose(0, 2, 1, 3), pad1
                      ).reshape(H, 2, S * 33)
    vhalves = jnp.pad(v.reshape(H, S, 2, 32).transpose(0, 2, 1, 3), pad1
                      ).reshape(H, 2, S * 33)
    qflat = jnp.pad(q, ((0, 0), (0, 0), (0, 1))).reshape(H, S * 65)

    def select_group(qg):
        return pl.pallas_call(
            _select_kernel,
            grid=(HG, S // BQ),
            in_specs=[pl.BlockSpec((1, S, DH), lambda h, i: (h, 0, 0))],
            out_specs=pl.BlockSpec((1, BQ, LS), lambda h, i: (h, i, 0)),
            out_shape=jax.ShapeDtypeStruct((HG, S, LS), jnp.int32),
        )(qg)

    outs = []
    for g0 in range(0, H, HG):
        idx_g = select_group(q[g0:g0 + HG])
        idxflat_g = jnp.pad(idx_g, ((0, 0), (0, 0), (0, 1))).reshape(HG, S * 17)
        outs.append(_sc_attention(khalves[g0:g0 + HG], vhalves[g0:g0 + HG],
                                  qflat[g0:g0 + HG], idxflat_g))
    out_sc = jnp.concatenate(outs, axis=0)

    y = (out_sc.reshape(H, 2, S, 33)[..., :32]
         .transpose(2, 0, 1, 3).reshape(S, D))

    res = pl.pallas_call(
        _out_proj_kernel,
        grid=(S // BQ,),
        in_specs=[
            pl.BlockSpec((BQ, D), lambda i: (i, 0)),
            pl.BlockSpec((D, D), lambda i: (0, 0)),
            pl.BlockSpec((D,), lambda i: (0,)),
            pl.BlockSpec((BQ, D), lambda i: (i, 0)),
        ],
        out_specs=pl.BlockSpec((BQ, D), lambda i: (i, 0)),
        out_shape=jax.ShapeDtypeStruct((S, D), jnp.float32),
    )(y, W_proj, b_proj, x2)

    return res.reshape(B, S, D)
